# baseline (speedup 1.0000x reference)
import functools

import jax
import jax.numpy as jnp
import numpy as np
from jax import lax
from jax.experimental import pallas as pl
from jax.experimental.pallas import tpu as pltpu
from jax.experimental.pallas import tpu_sc as plsc

N = 10000
E = 320000
D = 128
NC = 2
NS = 16
NW = NC * NS
NP = 10240
RPT = NP // NS
E2 = 327680
EPW = E2 // NW
CH = 64
NCHUNK = EPW // CH

_mesh = plsc.VectorSubcoreMesh(
    core_axis_name="c", subcore_axis_name="s", num_cores=NC, num_subcores=NS)


@functools.partial(
    pl.kernel,
    out_type=[
        jax.ShapeDtypeStruct((NC, NP, D), jnp.float32),
        jax.ShapeDtypeStruct((NC, NP), jnp.float32),
    ],
    mesh=_mesh,
    scratch_types=[
        pltpu.VMEM((NCHUNK // 2, CH), jnp.int32),
        pltpu.VMEM((NCHUNK // 2, CH), jnp.int32),
        [pltpu.VMEM((CH, D), jnp.float32) for _ in range(3)],
        pltpu.VMEM((CH,), jnp.float32),
        pltpu.VMEM((RPT,), jnp.float32),
        pltpu.VMEM_SHARED((NP, D), jnp.float32),
        pltpu.VMEM_SHARED((NP,), jnp.float32),
        [pltpu.SemaphoreType.DMA for _ in range(3)],
        [pltpu.SemaphoreType.DMA for _ in range(3)],
        pltpu.SemaphoreType.DMA,
    ],
)
def _spmm_kernel(x_hbm, e_hbm, y_hbm, cnt_hbm,
                 cidxh, ridxh, bufs, ones_v, zbuf, acc_sh, hist_sh,
                 sem_g, sem_s, sem_h):
    c = lax.axis_index("c")
    t = lax.axis_index("s")
    w = c * NS + t
    HC = NCHUNK // 2
    one16 = jnp.ones((16,), jnp.float32)
    z16 = jnp.zeros((16,), jnp.float32)
    for i in range(CH // 16):
        ones_v[pl.ds(16 * i, 16)] = one16

    def hzinit(i, _):
        zbuf[pl.ds(i * 16, 16)] = z16
        return 0
    lax.fori_loop(0, RPT // 16, hzinit, 0)
    pltpu.sync_copy(zbuf, hist_sh.at[pl.ds(t * RPT, RPT)])

    def zinit(i, _):
        bufs[0][i // 8, pl.ds(16 * (i % 8), 16)] = z16
        return 0
    lax.fori_loop(0, CH * D // 16, zinit, 0)
    for k in range(RPT // CH):
        pltpu.sync_copy(bufs[0], acc_sh.at[pl.ds(t * RPT + k * CH, CH)])
    plsc.subcore_barrier()

    for half in range(2):
        cb = w * NCHUNK + half * HC
        pltpu.sync_copy(e_hbm.at[1, pl.ds(cb, HC)], cidxh)
        pltpu.sync_copy(e_hbm.at[0, pl.ds(cb, HC)], ridxh)
        pltpu.async_copy(x_hbm.at[cidxh.at[0]], bufs[0], sem_g[0])
        pltpu.async_copy(x_hbm.at[cidxh.at[1]], bufs[1], sem_g[1])

        def hfire(i, _):
            pltpu.async_copy(ones_v, hist_sh.at[ridxh.at[i]], sem_h, add=True)
            return 0
        lax.fori_loop(0, HC, hfire, 0)

        def chunk(i, _):
            def step(b):
                buf = bufs[b]
                pltpu.make_async_copy(x_hbm.at[pl.ds(0, CH)], buf,
                                      sem_g[b]).wait()
                pltpu.async_copy(buf, acc_sh.at[ridxh.at[i]], sem_s[b],
                                 add=True)
                b2 = (b + 2) % 3

                @pl.when(i >= 1)
                def _():
                    pltpu.make_async_copy(bufs[b2], acc_sh.at[ridxh.at[i]],
                                          sem_s[b2]).wait()

                @pl.when(i + 2 < HC)
                def _():
                    pltpu.async_copy(x_hbm.at[cidxh.at[i + 2]], bufs[b2],
                                     sem_g[b2])

            for b in range(3):
                @pl.when(i % 3 == b)
                def _(b=b):
                    step(b)
            return 0
        lax.fori_loop(0, HC, chunk, 0)
        pltpu.make_async_copy(bufs[(HC - 1) % 3], acc_sh.at[ridxh.at[0]],
                              sem_s[(HC - 1) % 3]).wait()

        def hdrain(i, _):
            pltpu.make_async_copy(ones_v, hist_sh.at[ridxh.at[0]],
                                  sem_h).wait()
            return 0
        lax.fori_loop(0, HC, hdrain, 0)
    plsc.subcore_barrier()
    pltpu.sync_copy(acc_sh.at[pl.ds(t * RPT, RPT)],
                    y_hbm.at[c, pl.ds(t * RPT, RPT)])
    pltpu.sync_copy(hist_sh.at[pl.ds(t * RPT, RPT)],
                    cnt_hbm.at[c, pl.ds(t * RPT, RPT)])


_BR = 2000

_DIS_C = 0.7071067811865476


def _combine_body(cnt_ref, y_ref, x_ref, h_ref):
    cnt = cnt_ref[0] + cnt_ref[1]
    deg = cnt + 1.0
    inv = 1.0 / (deg + 1e-8)
    deg2 = deg * inv + 1.0
    dis = lax.rsqrt(deg2 + 1e-8)
    a2 = dis * inv * _DIS_C
    b = dis * dis * (inv + 1.0)
    y = y_ref[0] + y_ref[1]
    h_ref[...] = a2 * y + b * x_ref[...]


_combine_call = pl.pallas_call(
    _combine_body,
    grid=(N // _BR,),
    in_specs=[
        pl.BlockSpec((NC, _BR, 1), lambda i: (0, i, 0)),
        pl.BlockSpec((NC, _BR, D), lambda i: (0, i, 0)),
        pl.BlockSpec((_BR, D), lambda i: (i, 0)),
    ],
    out_specs=pl.BlockSpec((_BR, D), lambda i: (i, 0)),
    out_shape=jax.ShapeDtypeStruct((N, D), jnp.float32),
)

_PAD_R = np.asarray(
    N + (np.arange(E2 - E, dtype=np.int32) % (NP - N)), dtype=np.int32)
_PAD_C = np.asarray(np.arange(E2 - E, dtype=np.int32) % N, dtype=np.int32)


def kernel(x, edge_index):
    ep = jnp.concatenate([edge_index, jnp.stack([_PAD_R, _PAD_C])], axis=1)
    y, cnt = _spmm_kernel(x, ep.reshape(2, E2 // CH, CH))
    h = _combine_call(cnt.reshape(NC, NP, 1), y, x)
    return h


# speedup vs baseline: 1.0187x; 1.0187x over previous
import functools

import jax
import jax.numpy as jnp
import numpy as np
from jax import lax
from jax.experimental import pallas as pl
from jax.experimental.pallas import tpu as pltpu
from jax.experimental.pallas import tpu_sc as plsc

N = 10000
E = 320000
D = 128
NC = 2
NS = 16
NW = NC * NS
NP = 10240
RPT = NP // NS
CHH = 128
E2 = 327680
EPW = E2 // NW
NCHUNKH = EPW // CHH
CH = 64
NCHUNK = EPW // CH

_mesh = plsc.VectorSubcoreMesh(
    core_axis_name="c", subcore_axis_name="s", num_cores=NC, num_subcores=NS)


@functools.partial(
    pl.kernel,
    out_type=jax.ShapeDtypeStruct((NC, NP), jnp.float32),
    mesh=_mesh,
    scratch_types=[
        pltpu.VMEM((NCHUNKH, CHH), jnp.int32),
        pltpu.VMEM((CHH,), jnp.float32),
        pltpu.VMEM((RPT,), jnp.float32),
        pltpu.VMEM_SHARED((NP,), jnp.float32),
        pltpu.SemaphoreType.DMA,
    ],
)
def _hist_kernel(eh_hbm, cnt_hbm, idx2, ones_v, zbuf, hist_sh, sem):
    c = lax.axis_index("c")
    t = lax.axis_index("s")
    one16 = jnp.ones((16,), jnp.float32)
    zero16 = jnp.zeros((16,), jnp.float32)
    for i in range(8):
        ones_v[pl.ds(16 * i, 16)] = one16

    def zinit(i, _):
        zbuf[pl.ds(i * 16, 16)] = zero16
        return 0
    lax.fori_loop(0, RPT // 16, zinit, 0)
    pltpu.sync_copy(zbuf, hist_sh.at[pl.ds(t * RPT, RPT)])
    w = c * NS + t
    pltpu.sync_copy(eh_hbm.at[0, pl.ds(w * NCHUNKH, NCHUNKH)], idx2)
    plsc.subcore_barrier()

    def fire(i, _):
        pltpu.async_copy(ones_v, hist_sh.at[idx2.at[i]], sem, add=True)
        return 0
    lax.fori_loop(0, NCHUNKH, fire, 0)

    def drain(i, _):
        pltpu.make_async_copy(ones_v, hist_sh.at[idx2.at[0]], sem).wait()
        return 0
    lax.fori_loop(0, NCHUNKH, drain, 0)
    plsc.subcore_barrier()
    pltpu.sync_copy(hist_sh.at[pl.ds(t * RPT, RPT)],
                    cnt_hbm.at[c, pl.ds(t * RPT, RPT)])


@functools.partial(
    pl.kernel,
    out_type=jax.ShapeDtypeStruct((NC, NP, D), jnp.float32),
    mesh=_mesh,
    scratch_types=[
        pltpu.VMEM((NCHUNK // 2, CH), jnp.int32),
        pltpu.VMEM((NCHUNK // 2, CH), jnp.int32),
        [pltpu.VMEM((CH, D), jnp.float32) for _ in range(3)],
        pltpu.VMEM_SHARED((NP, D), jnp.float32),
        [pltpu.SemaphoreType.DMA for _ in range(3)],
        [pltpu.SemaphoreType.DMA for _ in range(3)],
    ],
)
def _spmm_kernel(x_hbm, e_hbm, y_hbm,
                 cidxh, ridxh, bufs, acc_sh, sem_g, sem_s):
    c = lax.axis_index("c")
    t = lax.axis_index("s")
    w = c * NS + t
    HC = NCHUNK // 2
    z16 = jnp.zeros((16,), jnp.float32)

    def zinit(i, _):
        bufs[0][i // 8, pl.ds(16 * (i % 8), 16)] = z16
        return 0
    lax.fori_loop(0, CH * D // 16, zinit, 0)
    for k in range(RPT // CH):
        pltpu.sync_copy(bufs[0], acc_sh.at[pl.ds(t * RPT + k * CH, CH)])
    plsc.subcore_barrier()

    for half in range(2):
        cb = w * NCHUNK + half * HC
        pltpu.sync_copy(e_hbm.at[1, pl.ds(cb, HC)], cidxh)
        pltpu.sync_copy(e_hbm.at[0, pl.ds(cb, HC)], ridxh)
        pltpu.async_copy(x_hbm.at[cidxh.at[0]], bufs[0], sem_g[0])
        pltpu.async_copy(x_hbm.at[cidxh.at[1]], bufs[1], sem_g[1])

        def chunk(i, _):
            def step(b):
                buf = bufs[b]
                pltpu.make_async_copy(x_hbm.at[pl.ds(0, CH)], buf,
                                      sem_g[b]).wait()
                pltpu.async_copy(buf, acc_sh.at[ridxh.at[i]], sem_s[b],
                                 add=True)
                b2 = (b + 2) % 3

                @pl.when(i >= 1)
                def _():
                    pltpu.make_async_copy(bufs[b2], acc_sh.at[ridxh.at[i]],
                                          sem_s[b2]).wait()

                @pl.when(i + 2 < HC)
                def _():
                    pltpu.async_copy(x_hbm.at[cidxh.at[i + 2]], bufs[b2],
                                     sem_g[b2])

            for b in range(3):
                @pl.when(i % 3 == b)
                def _(b=b):
                    step(b)
            return 0
        lax.fori_loop(0, HC, chunk, 0)
        pltpu.make_async_copy(bufs[(HC - 1) % 3], acc_sh.at[ridxh.at[0]],
                              sem_s[(HC - 1) % 3]).wait()
    plsc.subcore_barrier()
    pltpu.sync_copy(acc_sh.at[pl.ds(t * RPT, RPT)],
                    y_hbm.at[c, pl.ds(t * RPT, RPT)])


_BR = 2000

_DIS_C = 0.7071067811865476


def _combine_body(cnt_ref, y_ref, x_ref, h_ref):
    cnt = cnt_ref[0] + cnt_ref[1]
    deg = cnt + 1.0
    inv = 1.0 / (deg + 1e-8)
    deg2 = deg * inv + 1.0
    dis = lax.rsqrt(deg2 + 1e-8)
    a2 = dis * inv * _DIS_C
    b = dis * dis * (inv + 1.0)
    y = y_ref[0] + y_ref[1]
    h_ref[...] = a2 * y + b * x_ref[...]


_combine_call = pl.pallas_call(
    _combine_body,
    grid=(N // _BR,),
    in_specs=[
        pl.BlockSpec((NC, _BR, 1), lambda i: (0, i, 0)),
        pl.BlockSpec((NC, _BR, D), lambda i: (0, i, 0)),
        pl.BlockSpec((_BR, D), lambda i: (i, 0)),
    ],
    out_specs=pl.BlockSpec((_BR, D), lambda i: (i, 0)),
    out_shape=jax.ShapeDtypeStruct((N, D), jnp.float32),
)

_PAD_R = np.asarray(
    N + (np.arange(E2 - E, dtype=np.int32) % (NP - N)), dtype=np.int32)
_PAD_C = np.asarray(np.arange(E2 - E, dtype=np.int32) % N, dtype=np.int32)


def kernel(x, edge_index):
    ep = jnp.concatenate([edge_index, jnp.stack([_PAD_R, _PAD_C])], axis=1)
    cnt = _hist_kernel(ep.reshape(2, E2 // CHH, CHH))
    y = _spmm_kernel(x, ep.reshape(2, E2 // CH, CH))
    h = _combine_call(cnt.reshape(NC, NP, 1), y, x)
    return h
